# Initial kernel scaffold; baseline (speedup 1.0000x reference)
import functools

import jax
import jax.numpy as jnp
from jax import lax
from jax.experimental import pallas as pl
from jax.experimental.pallas import tpu as pltpu
from jax.experimental.pallas import tpu_sc as plsc

_N = 10000
_E = 320000
_D = 128
_NC = 2
_NS = 16
_NW = _NC * _NS
_CH = 80
_NBALL = _E // _CH
_CPW = _NBALL // _NW
_NP = 10240
_NPW = _NP // _NS
_R = 2000

_mesh = plsc.VectorSubcoreMesh(
    core_axis_name="c", subcore_axis_name="s", num_cores=_NC, num_subcores=_NS)


def _deg_body(ed_hbm, out_hbm, ebuf, ones_v, zbuf, acc_sh):
    c = lax.axis_index("c")
    s = lax.axis_index("s")
    wid = c * _NS + s

    def zb(i, _):
        zbuf[pl.ds(i * 16, 16)] = jnp.zeros((16,), jnp.float32)
        return 0
    lax.fori_loop(0, _NPW // 16, zb, 0)

    def ob(i, _):
        ones_v[pl.ds(i * 16, 16)] = jnp.ones((16,), jnp.float32)
        return 0
    lax.fori_loop(0, _CH // 16, ob, 0)

    pltpu.sync_copy(zbuf, acc_sh.at[pl.ds(s * _NPW, _NPW)])
    plsc.subcore_barrier()

    def batch(g, _):
        gi = wid * _CPW + g
        pltpu.sync_copy(ed_hbm.at[gi, 0], ebuf)
        pltpu.sync_copy(ones_v, acc_sh.at[ebuf], add=True)
        return 0
    lax.fori_loop(0, _CPW, batch, 0)

    plsc.subcore_barrier()
    pltpu.sync_copy(acc_sh.at[pl.ds(s * _NPW, _NPW)],
                    out_hbm.at[c, pl.ds(s * _NPW, _NPW)])


_deg_call = functools.partial(
    pl.kernel,
    out_type=jax.ShapeDtypeStruct((_NC, _NP), jnp.float32),
    mesh=_mesh,
    scratch_types=[
        pltpu.VMEM((_CH,), jnp.int32),
        pltpu.VMEM((_CH,), jnp.float32),
        pltpu.VMEM((_NPW,), jnp.float32),
        pltpu.VMEM_SHARED((_NP,), jnp.float32),
    ],
)(_deg_body)


def _dense_body(x_ref, wt_ref, b_ref, dps_ref, b0_ref, b1_ref, b2_ref,
                xs_ref, xg_ref, dinv_ref, c_ref):
    xl = jnp.dot(x_ref[...], wt_ref[...], preferred_element_type=jnp.float32)
    xl = xl + b_ref[...]
    deg = dps_ref[...] + 1.0
    dinv = lax.rsqrt(deg)
    xs_ref[...] = xl
    xg_ref[...] = xl * dinv
    dinv_ref[...] = dinv
    t01 = (jnp.broadcast_to(b0_ref[...][:, None, :], (8, 8, _D))
           + b1_ref[...][None, :, :]).reshape(64, _D)
    c_ref[...] = (jnp.broadcast_to(t01[:, None, :], (64, 8, _D))
                  + b2_ref[...][None, :, :]).reshape(512, _D)


def _dense_call(x, wt, bvec, dps, b0, b1, b2):
    return pl.pallas_call(
        _dense_body,
        grid=(_N // _R,),
        in_specs=[
            pl.BlockSpec((_R, _D), lambda i: (i, 0)),
            pl.BlockSpec((_D, _D), lambda i: (0, 0)),
            pl.BlockSpec((1, _D), lambda i: (0, 0)),
            pl.BlockSpec((_R, 1), lambda i: (i, 0)),
            pl.BlockSpec((8, _D), lambda i: (0, 0)),
            pl.BlockSpec((8, _D), lambda i: (0, 0)),
            pl.BlockSpec((8, _D), lambda i: (0, 0)),
        ],
        out_specs=[
            pl.BlockSpec((_R, _D), lambda i: (i, 0)),
            pl.BlockSpec((_R, _D), lambda i: (i, 0)),
            pl.BlockSpec((_R, 1), lambda i: (i, 0)),
            pl.BlockSpec((512, _D), lambda i: (0, 0)),
        ],
        out_shape=[
            jax.ShapeDtypeStruct((_N, _D), jnp.float32),
            jax.ShapeDtypeStruct((_N, _D), jnp.float32),
            jax.ShapeDtypeStruct((_N, 1), jnp.float32),
            jax.ShapeDtypeStruct((512, _D), jnp.float32),
        ],
    )(x, wt, bvec, dps, b0, b1, b2)


def _main_body(ed_hbm, xg_hbm, dinv_hbm, c_hbm, parts_hbm,
               ebuf, cidv, dvb, xbuf, cbuf, acc_sh):
    c = lax.axis_index("c")
    s = lax.axis_index("s")
    wid = c * _NS + s

    pltpu.sync_copy(c_hbm, cbuf)

    def zb(i, _):
        r = i // 8
        k = i % 8
        xbuf[r, pl.ds(k * 16, 16)] = jnp.zeros((16,), jnp.float32)
        return 0
    lax.fori_loop(0, _CH * (_D // 16), zb, 0)

    def zc(j, _):
        pltpu.sync_copy(xbuf, acc_sh.at[pl.ds(s * _NPW + j * _CH, _CH)])
        return 0
    lax.fori_loop(0, _NPW // _CH, zc, 0)
    plsc.subcore_barrier()

    def batch(g, _):
        gi = wid * _CPW + g
        pltpu.sync_copy(ed_hbm.at[gi], ebuf)

        def cidb(j, _):
            sl = pl.ds(j * 16, 16)
            cidv[sl] = (ebuf[2, sl] * 8 + ebuf[3, sl]) * 8 + ebuf[4, sl]
            return 0
        lax.fori_loop(0, _CH // 16, cidb, 0)

        pltpu.sync_copy(xg_hbm.at[ebuf.at[0]], xbuf)
        pltpu.sync_copy(dinv_hbm.at[ebuf.at[0]], dvb)

        def edge(bi, _):
            cid = cidv[bi]
            dv = dvb[bi]
            for k in range(_D // 16):
                sl = pl.ds(k * 16, 16)
                xbuf[bi, sl] = jnp.maximum(xbuf[bi, sl] + dv * cbuf[cid, sl],
                                           0.0)
            return 0
        lax.fori_loop(0, _CH, edge, 0)

        pltpu.sync_copy(xbuf, acc_sh.at[ebuf.at[1]], add=True)
        return 0
    lax.fori_loop(0, _CPW, batch, 0)

    plsc.subcore_barrier()
    pltpu.sync_copy(acc_sh.at[pl.ds(s * _NPW, _NPW)],
                    parts_hbm.at[c, pl.ds(s * _NPW, _NPW)])


_main_call = functools.partial(
    pl.kernel,
    out_type=jax.ShapeDtypeStruct((_NC, _NP, _D), jnp.float32),
    mesh=_mesh,
    scratch_types=[
        pltpu.VMEM((5, _CH), jnp.int32),
        pltpu.VMEM((_CH,), jnp.int32),
        pltpu.VMEM((_CH,), jnp.float32),
        pltpu.VMEM((_CH, _D), jnp.float32),
        pltpu.VMEM((512, _D), jnp.float32),
        pltpu.VMEM_SHARED((_NP, _D), jnp.float32),
    ],
)(_main_body)


def _final_body(p0_ref, p1_ref, xs_ref, root_ref, dinv_ref, out_ref):
    dinv = dinv_ref[...]
    selfv = jnp.maximum(xs_ref[...] + root_ref[...], 0.0) * (dinv * dinv)
    out_ref[...] = (p0_ref[...] + p1_ref[...]) * dinv + selfv


def _final_call(p0, p1, xs, root, dinv2):
    return pl.pallas_call(
        _final_body,
        grid=(_N // _R,),
        in_specs=[
            pl.BlockSpec((_R, _D), lambda i: (i, 0)),
            pl.BlockSpec((_R, _D), lambda i: (i, 0)),
            pl.BlockSpec((_R, _D), lambda i: (i, 0)),
            pl.BlockSpec((1, _D), lambda i: (0, 0)),
            pl.BlockSpec((_R, 1), lambda i: (i, 0)),
        ],
        out_specs=pl.BlockSpec((_R, _D), lambda i: (i, 0)),
        out_shape=jax.ShapeDtypeStruct((_N, _D), jnp.float32),
    )(p0, p1, xs, root, dinv2)


def kernel(x, edge_index, edge_attr, W, b, root_emb, B0, B1, B2):
    row = edge_index[0]
    col = edge_index[1]
    packed = jnp.stack(
        [row, col, edge_attr[:, 0], edge_attr[:, 1], edge_attr[:, 2]], 0)
    packed = packed.reshape(5, _NBALL, _CH).transpose(1, 0, 2)

    degp = _deg_call(packed)
    dps = (degp[0, :_N] + degp[1, :_N]).reshape(_N, 1)

    xs, xg, dinv2, C = _dense_call(
        x, W.T, b.reshape(1, _D), dps, B0, B1, B2)

    parts = _main_call(packed, xg, dinv2.reshape(_N), C)

    return _final_call(parts[0, :_N], parts[1, :_N], xs, root_emb, dinv2)


# speedup vs baseline: 4.7231x; 4.7231x over previous
import functools

import jax
import jax.numpy as jnp
from jax import lax
from jax.experimental import pallas as pl
from jax.experimental.pallas import tpu as pltpu
from jax.experimental.pallas import tpu_sc as plsc

_N = 10000
_E = 320000
_D = 128
_NC = 2
_NS = 16
_NW = _NC * _NS
_CH = 80
_NBALL = _E // _CH
_CPW = _NBALL // _NW
_NP = 10240
_NPW = _NP // _NS
_R = 2000

_HALF = 5120
_HP = 5248
_HPW = _HP // _NS
_CPS = _NBALL // _NS

_mesh = plsc.VectorSubcoreMesh(
    core_axis_name="c", subcore_axis_name="s", num_cores=_NC, num_subcores=_NS)


def _deg_body(ed_hbm, out_hbm, ebuf, ones_v, zbuf, acc_sh):
    c = lax.axis_index("c")
    s = lax.axis_index("s")
    wid = c * _NS + s

    def zb(i, _):
        zbuf[pl.ds(i * 16, 16)] = jnp.zeros((16,), jnp.float32)
        return 0
    lax.fori_loop(0, _NPW // 16, zb, 0)

    def ob(i, _):
        ones_v[pl.ds(i * 16, 16)] = jnp.ones((16,), jnp.float32)
        return 0
    lax.fori_loop(0, _CH // 16, ob, 0)

    pltpu.sync_copy(zbuf, acc_sh.at[pl.ds(s * _NPW, _NPW)])
    plsc.subcore_barrier()

    def batch(g, _):
        gi = wid * _CPW + g
        pltpu.sync_copy(ed_hbm.at[gi, 0], ebuf)
        pltpu.sync_copy(ones_v, acc_sh.at[ebuf], add=True)
        return 0
    lax.fori_loop(0, _CPW, batch, 0)

    plsc.subcore_barrier()
    pltpu.sync_copy(acc_sh.at[pl.ds(s * _NPW, _NPW)],
                    out_hbm.at[c, pl.ds(s * _NPW, _NPW)])


_deg_call = functools.partial(
    pl.kernel,
    out_type=jax.ShapeDtypeStruct((_NC, _NP), jnp.float32),
    mesh=_mesh,
    scratch_types=[
        pltpu.VMEM((_CH,), jnp.int32),
        pltpu.VMEM((_CH,), jnp.float32),
        pltpu.VMEM((_NPW,), jnp.float32),
        pltpu.VMEM_SHARED((_NP,), jnp.float32),
    ],
)(_deg_body)


def _dense_body(x_ref, wt_ref, b_ref, dps_ref, b0_ref, b1_ref, b2_ref,
                xs_ref, xg_ref, dinv_ref, c_ref):
    xl = jnp.dot(x_ref[...], wt_ref[...], preferred_element_type=jnp.float32)
    xl = xl + b_ref[...]
    deg = dps_ref[...] + 1.0
    dinv = lax.rsqrt(deg)
    xs_ref[...] = xl
    xg_ref[...] = xl * dinv
    dinv_ref[...] = dinv
    t01 = (jnp.broadcast_to(b0_ref[...][:, None, :], (8, 8, _D))
           + b1_ref[...][None, :, :]).reshape(64, _D)
    c_ref[...] = (jnp.broadcast_to(t01[:, None, :], (64, 8, _D))
                  + b2_ref[...][None, :, :]).reshape(512, _D)


def _dense_call(x, wt, bvec, dps, b0, b1, b2):
    return pl.pallas_call(
        _dense_body,
        grid=(_N // _R,),
        in_specs=[
            pl.BlockSpec((_R, _D), lambda i: (i, 0)),
            pl.BlockSpec((_D, _D), lambda i: (0, 0)),
            pl.BlockSpec((1, _D), lambda i: (0, 0)),
            pl.BlockSpec((_R, 1), lambda i: (i, 0)),
            pl.BlockSpec((8, _D), lambda i: (0, 0)),
            pl.BlockSpec((8, _D), lambda i: (0, 0)),
            pl.BlockSpec((8, _D), lambda i: (0, 0)),
        ],
        out_specs=[
            pl.BlockSpec((_R, _D), lambda i: (i, 0)),
            pl.BlockSpec((_R, _D), lambda i: (i, 0)),
            pl.BlockSpec((_R, 1), lambda i: (i, 0)),
            pl.BlockSpec((512, _D), lambda i: (0, 0)),
        ],
        out_shape=[
            jax.ShapeDtypeStruct((_N, _D), jnp.float32),
            jax.ShapeDtypeStruct((_N, _D), jnp.float32),
            jax.ShapeDtypeStruct((_N, 1), jnp.float32),
            jax.ShapeDtypeStruct((512, _D), jnp.float32),
        ],
    )(x, wt, bvec, dps, b0, b1, b2)


def _main_body(ed_hbm, xg_hbm, dinv_hbm, c_hbm, parts_hbm,
               ebuf, cidv, lcv, dvb, xbuf, cbuf, acc_sh):
    c = lax.axis_index("c")
    s = lax.axis_index("s")
    base_node = c * _HALF

    pltpu.sync_copy(c_hbm, cbuf)

    def zb(i, _):
        r = i // 8
        k = i % 8
        xbuf[r, pl.ds(k * 16, 16)] = jnp.zeros((16,), jnp.float32)
        return 0
    lax.fori_loop(0, _CH * (_D // 16), zb, 0)

    def zc(j, _):
        pltpu.sync_copy(xbuf, acc_sh.at[pl.ds(s * _HPW + j * _CH, _CH)])
        return 0
    lax.fori_loop(0, _HPW // _CH, zc, 0)
    pltpu.sync_copy(xbuf.at[pl.ds(0, _HPW % _CH)],
                    acc_sh.at[pl.ds(s * _HPW + (_HPW // _CH) * _CH,
                                    _HPW % _CH)])
    plsc.subcore_barrier()

    def batch(g, _):
        gi = s * _CPS + g
        pltpu.sync_copy(ed_hbm.at[gi], ebuf)

        def cidb(j, _):
            sl = pl.ds(j * 16, 16)
            cidv[sl] = (ebuf[2, sl] * 8 + ebuf[3, sl]) * 8 + ebuf[4, sl]
            lc = ebuf[1, sl] - base_node
            ok = (lc >= 0) & (lc < _HALF)
            lcv[sl] = jnp.where(ok, lc, _HALF)
            return 0
        lax.fori_loop(0, _CH // 16, cidb, 0)

        pltpu.sync_copy(xg_hbm.at[ebuf.at[0]], xbuf)
        pltpu.sync_copy(dinv_hbm.at[ebuf.at[0]], dvb)

        def edge16(j, _):
            cid16 = cidv[pl.ds(j * 16, 16)]
            dv16 = dvb[pl.ds(j * 16, 16)]
            for lane in range(16):
                bi = j * 16 + lane
                cid = cid16[lane]
                dv = dv16[lane]
                for k in range(_D // 16):
                    sl = pl.ds(k * 16, 16)
                    xbuf[bi, sl] = jnp.maximum(
                        xbuf[bi, sl] + dv * cbuf[cid, sl], 0.0)
            return 0
        lax.fori_loop(0, _CH // 16, edge16, 0)

        pltpu.sync_copy(xbuf, acc_sh.at[lcv], add=True)
        return 0
    lax.fori_loop(0, _CPS, batch, 0)

    plsc.subcore_barrier()
    pltpu.sync_copy(acc_sh.at[pl.ds(s * _HPW, _HPW)],
                    parts_hbm.at[c, pl.ds(s * _HPW, _HPW)])


_main_call = functools.partial(
    pl.kernel,
    out_type=jax.ShapeDtypeStruct((_NC, _HP, _D), jnp.float32),
    mesh=_mesh,
    scratch_types=[
        pltpu.VMEM((5, _CH), jnp.int32),
        pltpu.VMEM((_CH,), jnp.int32),
        pltpu.VMEM((_CH,), jnp.int32),
        pltpu.VMEM((_CH,), jnp.float32),
        pltpu.VMEM((_CH, _D), jnp.float32),
        pltpu.VMEM((512, _D), jnp.float32),
        pltpu.VMEM_SHARED((_HP, _D), jnp.float32),
    ],
)(_main_body)


def _final_body(p_ref, xs_ref, root_ref, dinv_ref, out_ref):
    dinv = dinv_ref[...]
    selfv = jnp.maximum(xs_ref[...] + root_ref[...], 0.0) * (dinv * dinv)
    out_ref[...] = p_ref[...] * dinv + selfv


def _final_call(pfull, xs, root, dinv2):
    return pl.pallas_call(
        _final_body,
        grid=(_N // _R,),
        in_specs=[
            pl.BlockSpec((_R, _D), lambda i: (i, 0)),
            pl.BlockSpec((_R, _D), lambda i: (i, 0)),
            pl.BlockSpec((1, _D), lambda i: (0, 0)),
            pl.BlockSpec((_R, 1), lambda i: (i, 0)),
        ],
        out_specs=pl.BlockSpec((_R, _D), lambda i: (i, 0)),
        out_shape=jax.ShapeDtypeStruct((_N, _D), jnp.float32),
    )(pfull, xs, root, dinv2)


def kernel(x, edge_index, edge_attr, W, b, root_emb, B0, B1, B2):
    row = edge_index[0]
    col = edge_index[1]
    packed = jnp.stack(
        [row, col, edge_attr[:, 0], edge_attr[:, 1], edge_attr[:, 2]], 0)
    packed = packed.reshape(5, _NBALL, _CH).transpose(1, 0, 2)

    degp = _deg_call(packed)
    dps = (degp[0, :_N] + degp[1, :_N]).reshape(_N, 1)

    xs, xg, dinv2, C = _dense_call(
        x, W.T, b.reshape(1, _D), dps, B0, B1, B2)

    parts = _main_call(packed, xg, dinv2.reshape(_N), C)
    pfull = jnp.concatenate(
        [parts[0, :_HALF], parts[1, :_N - _HALF]], axis=0)

    return _final_call(pfull, xs, root_emb, dinv2)


# speedup vs baseline: 5.0394x; 1.0670x over previous
import functools

import jax
import jax.numpy as jnp
from jax import lax
from jax.experimental import pallas as pl
from jax.experimental.pallas import tpu as pltpu
from jax.experimental.pallas import tpu_sc as plsc

_N = 10000
_E = 320000
_D = 128
_NC = 2
_NS = 16
_NW = _NC * _NS
_CH = 80
_NBALL = _E // _CH
_CPW = _NBALL // _NW
_NP = 10240
_NPW = _NP // _NS
_R = 2000

_HALF = 5120
_HP = 5248
_HPW = _HP // _NS
_CPS = _NBALL // _NS

_mesh = plsc.VectorSubcoreMesh(
    core_axis_name="c", subcore_axis_name="s", num_cores=_NC, num_subcores=_NS)


def _deg_body(ed_hbm, out_hbm, ebuf, ones_v, zbuf, acc_sh):
    c = lax.axis_index("c")
    s = lax.axis_index("s")
    wid = c * _NS + s

    def zb(i, _):
        zbuf[pl.ds(i * 16, 16)] = jnp.zeros((16,), jnp.float32)
        return 0
    lax.fori_loop(0, _NPW // 16, zb, 0)

    def ob(i, _):
        ones_v[pl.ds(i * 16, 16)] = jnp.ones((16,), jnp.float32)
        return 0
    lax.fori_loop(0, _CH // 16, ob, 0)

    pltpu.sync_copy(zbuf, acc_sh.at[pl.ds(s * _NPW, _NPW)])
    plsc.subcore_barrier()

    def batch(g, _):
        gi = wid * _CPW + g
        pltpu.sync_copy(ed_hbm.at[gi, 0], ebuf)
        pltpu.sync_copy(ones_v, acc_sh.at[ebuf], add=True)
        return 0
    lax.fori_loop(0, _CPW, batch, 0)

    plsc.subcore_barrier()
    pltpu.sync_copy(acc_sh.at[pl.ds(s * _NPW, _NPW)],
                    out_hbm.at[c, pl.ds(s * _NPW, _NPW)])


_deg_call = functools.partial(
    pl.kernel,
    out_type=jax.ShapeDtypeStruct((_NC, _NP), jnp.float32),
    mesh=_mesh,
    scratch_types=[
        pltpu.VMEM((_CH,), jnp.int32),
        pltpu.VMEM((_CH,), jnp.float32),
        pltpu.VMEM((_NPW,), jnp.float32),
        pltpu.VMEM_SHARED((_NP,), jnp.float32),
    ],
)(_deg_body)


def _dense_body(x_ref, wt_ref, b_ref, dps_ref, b0_ref, b1_ref, b2_ref,
                xs_ref, xg_ref, dinv_ref, c_ref):
    xl = jnp.dot(x_ref[...], wt_ref[...], preferred_element_type=jnp.float32)
    xl = xl + b_ref[...]
    deg = dps_ref[...] + 1.0
    dinv = lax.rsqrt(deg)
    xs_ref[...] = xl
    xg_ref[...] = xl * dinv
    dinv_ref[...] = dinv
    t01 = (jnp.broadcast_to(b0_ref[...][:, None, :], (8, 8, _D))
           + b1_ref[...][None, :, :]).reshape(64, _D)
    c_ref[...] = (jnp.broadcast_to(t01[:, None, :], (64, 8, _D))
                  + b2_ref[...][None, :, :]).reshape(512, _D)


def _dense_call(x, wt, bvec, dps, b0, b1, b2):
    return pl.pallas_call(
        _dense_body,
        grid=(_N // _R,),
        in_specs=[
            pl.BlockSpec((_R, _D), lambda i: (i, 0)),
            pl.BlockSpec((_D, _D), lambda i: (0, 0)),
            pl.BlockSpec((1, _D), lambda i: (0, 0)),
            pl.BlockSpec((_R, 1), lambda i: (i, 0)),
            pl.BlockSpec((8, _D), lambda i: (0, 0)),
            pl.BlockSpec((8, _D), lambda i: (0, 0)),
            pl.BlockSpec((8, _D), lambda i: (0, 0)),
        ],
        out_specs=[
            pl.BlockSpec((_R, _D), lambda i: (i, 0)),
            pl.BlockSpec((_R, _D), lambda i: (i, 0)),
            pl.BlockSpec((_R, 1), lambda i: (i, 0)),
            pl.BlockSpec((512, _D), lambda i: (0, 0)),
        ],
        out_shape=[
            jax.ShapeDtypeStruct((_N, _D), jnp.float32),
            jax.ShapeDtypeStruct((_N, _D), jnp.float32),
            jax.ShapeDtypeStruct((_N, 1), jnp.float32),
            jax.ShapeDtypeStruct((512, _D), jnp.float32),
        ],
    )(x, wt, bvec, dps, b0, b1, b2)


def _main_body(ed_hbm, xg_hbm, dinv_hbm, c_hbm, parts_hbm,
               ebuf, cidv, lcv, dvb, xbuf, cbuf, acc_sh):
    c = lax.axis_index("c")
    s = lax.axis_index("s")
    base_node = c * _HALF

    pltpu.sync_copy(c_hbm, cbuf)

    def zb(i, _):
        r = i // 8
        k = i % 8
        xbuf[r, pl.ds(k * 16, 16)] = jnp.zeros((16,), jnp.float32)
        return 0
    lax.fori_loop(0, _CH * (_D // 16), zb, 0)

    def zc(j, _):
        pltpu.sync_copy(xbuf, acc_sh.at[pl.ds(s * _HPW + j * _CH, _CH)])
        return 0
    lax.fori_loop(0, _HPW // _CH, zc, 0)
    pltpu.sync_copy(xbuf.at[pl.ds(0, _HPW % _CH)],
                    acc_sh.at[pl.ds(s * _HPW + (_HPW // _CH) * _CH,
                                    _HPW % _CH)])
    plsc.subcore_barrier()

    def batch(g, _):
        gi = s * _CPS + g
        pltpu.sync_copy(ed_hbm.at[gi], ebuf)

        def cidb(j, _):
            sl = pl.ds(j * 16, 16)
            cidv[sl] = (ebuf[2, sl] * 8 + ebuf[3, sl]) * 8 + ebuf[4, sl]
            lc = ebuf[1, sl] - base_node
            ok = (lc >= 0) & (lc < _HALF)
            lcv[sl] = jnp.where(ok, lc, _HALF)
            return 0
        lax.fori_loop(0, _CH // 16, cidb, 0)

        pltpu.sync_copy(xg_hbm.at[ebuf.at[0]], xbuf)
        pltpu.sync_copy(dinv_hbm.at[ebuf.at[0]], dvb)
        @plsc.parallel_loop(0, _CH // 16)
        def _edge16(j):
            cid16 = cidv[pl.ds(j * 16, 16)]
            dv16 = dvb[pl.ds(j * 16, 16)]
            for lane in range(16):
                bi = j * 16 + lane
                cid = cid16[lane]
                dv = dv16[lane]
                for k in range(_D // 16):
                    sl = pl.ds(k * 16, 16)
                    xbuf[bi, sl] = jnp.maximum(
                        xbuf[bi, sl] + dv * cbuf[cid, sl], 0.0)

        pltpu.sync_copy(xbuf, acc_sh.at[lcv], add=True)
        return 0
    lax.fori_loop(0, _CPS, batch, 0)

    plsc.subcore_barrier()
    pltpu.sync_copy(acc_sh.at[pl.ds(s * _HPW, _HPW)],
                    parts_hbm.at[c, pl.ds(s * _HPW, _HPW)])


_main_call = functools.partial(
    pl.kernel,
    out_type=jax.ShapeDtypeStruct((_NC, _HP, _D), jnp.float32),
    mesh=_mesh,
    scratch_types=[
        pltpu.VMEM((5, _CH), jnp.int32),
        pltpu.VMEM((_CH,), jnp.int32),
        pltpu.VMEM((_CH,), jnp.int32),
        pltpu.VMEM((_CH,), jnp.float32),
        pltpu.VMEM((_CH, _D), jnp.float32),
        pltpu.VMEM((512, _D), jnp.float32),
        pltpu.VMEM_SHARED((_HP, _D), jnp.float32),
    ],
)(_main_body)


def _final_body(p_ref, xs_ref, root_ref, dinv_ref, out_ref):
    dinv = dinv_ref[...]
    selfv = jnp.maximum(xs_ref[...] + root_ref[...], 0.0) * (dinv * dinv)
    out_ref[...] = p_ref[...] * dinv + selfv


def _final_call(pfull, xs, root, dinv2):
    return pl.pallas_call(
        _final_body,
        grid=(_N // _R,),
        in_specs=[
            pl.BlockSpec((_R, _D), lambda i: (i, 0)),
            pl.BlockSpec((_R, _D), lambda i: (i, 0)),
            pl.BlockSpec((1, _D), lambda i: (0, 0)),
            pl.BlockSpec((_R, 1), lambda i: (i, 0)),
        ],
        out_specs=pl.BlockSpec((_R, _D), lambda i: (i, 0)),
        out_shape=jax.ShapeDtypeStruct((_N, _D), jnp.float32),
    )(pfull, xs, root, dinv2)


def kernel(x, edge_index, edge_attr, W, b, root_emb, B0, B1, B2):
    row = edge_index[0]
    col = edge_index[1]
    packed = jnp.stack(
        [row, col, edge_attr[:, 0], edge_attr[:, 1], edge_attr[:, 2]], 0)
    packed = packed.reshape(5, _NBALL, _CH).transpose(1, 0, 2)

    degp = _deg_call(packed)
    dps = (degp[0, :_N] + degp[1, :_N]).reshape(_N, 1)

    xs, xg, dinv2, C = _dense_call(
        x, W.T, b.reshape(1, _D), dps, B0, B1, B2)

    parts = _main_call(packed, xg, dinv2.reshape(_N), C)
    pfull = jnp.concatenate(
        [parts[0, :_HALF], parts[1, :_N - _HALF]], axis=0)

    return _final_call(pfull, xs, root_emb, dinv2)


# speedup vs baseline: 5.6400x; 1.1192x over previous
import functools

import jax
import jax.numpy as jnp
from jax import lax
from jax.experimental import pallas as pl
from jax.experimental.pallas import tpu as pltpu
from jax.experimental.pallas import tpu_sc as plsc

_N = 10000
_E = 320000
_D = 128
_NC = 2
_NS = 16
_NW = _NC * _NS
_CH = 80
_NBALL = _E // _CH
_CPW = _NBALL // _NW
_NP = 10240
_NPW = _NP // _NS
_R = 2000

_HALF = 5120
_HP = 5248
_HPW = _HP // _NS
_CPS = _NBALL // _NS

_mesh = plsc.VectorSubcoreMesh(
    core_axis_name="c", subcore_axis_name="s", num_cores=_NC, num_subcores=_NS)


def _deg_body(ed_hbm, out_hbm, ebuf, ones_v, zbuf, acc_sh):
    c = lax.axis_index("c")
    s = lax.axis_index("s")
    wid = c * _NS + s

    def zb(i, _):
        zbuf[pl.ds(i * 16, 16)] = jnp.zeros((16,), jnp.float32)
        return 0
    lax.fori_loop(0, _NPW // 16, zb, 0)

    def ob(i, _):
        ones_v[pl.ds(i * 16, 16)] = jnp.ones((16,), jnp.float32)
        return 0
    lax.fori_loop(0, _CH // 16, ob, 0)

    pltpu.sync_copy(zbuf, acc_sh.at[pl.ds(s * _NPW, _NPW)])
    plsc.subcore_barrier()

    def batch(g, _):
        gi = wid * _CPW + g
        pltpu.sync_copy(ed_hbm.at[gi, 0], ebuf)
        pltpu.sync_copy(ones_v, acc_sh.at[ebuf], add=True)
        return 0
    lax.fori_loop(0, _CPW, batch, 0)

    plsc.subcore_barrier()
    pltpu.sync_copy(acc_sh.at[pl.ds(s * _NPW, _NPW)],
                    out_hbm.at[c, pl.ds(s * _NPW, _NPW)])


_deg_call = functools.partial(
    pl.kernel,
    out_type=jax.ShapeDtypeStruct((_NC, _NP), jnp.float32),
    mesh=_mesh,
    scratch_types=[
        pltpu.VMEM((_CH,), jnp.int32),
        pltpu.VMEM((_CH,), jnp.float32),
        pltpu.VMEM((_NPW,), jnp.float32),
        pltpu.VMEM_SHARED((_NP,), jnp.float32),
    ],
)(_deg_body)


def _dense_body(x_ref, wt_ref, b_ref, dps_ref, b0_ref, b1_ref, b2_ref,
                xs_ref, xg_ref, dinv_ref, c_ref):
    xl = jnp.dot(x_ref[...], wt_ref[...], preferred_element_type=jnp.float32)
    xl = xl + b_ref[...]
    deg = dps_ref[...] + 1.0
    dinv = lax.rsqrt(deg)
    xs_ref[...] = xl
    xg_ref[...] = xl * dinv
    dinv_ref[...] = dinv
    t01 = (jnp.broadcast_to(b0_ref[...][:, None, :], (8, 8, _D))
           + b1_ref[...][None, :, :]).reshape(64, _D)
    c_ref[...] = (jnp.broadcast_to(t01[:, None, :], (64, 8, _D))
                  + b2_ref[...][None, :, :]).reshape(512, _D)


def _dense_call(x, wt, bvec, dps, b0, b1, b2):
    return pl.pallas_call(
        _dense_body,
        grid=(_N // _R,),
        in_specs=[
            pl.BlockSpec((_R, _D), lambda i: (i, 0)),
            pl.BlockSpec((_D, _D), lambda i: (0, 0)),
            pl.BlockSpec((1, _D), lambda i: (0, 0)),
            pl.BlockSpec((_R, 1), lambda i: (i, 0)),
            pl.BlockSpec((8, _D), lambda i: (0, 0)),
            pl.BlockSpec((8, _D), lambda i: (0, 0)),
            pl.BlockSpec((8, _D), lambda i: (0, 0)),
        ],
        out_specs=[
            pl.BlockSpec((_R, _D), lambda i: (i, 0)),
            pl.BlockSpec((_R, _D), lambda i: (i, 0)),
            pl.BlockSpec((_R, 1), lambda i: (i, 0)),
            pl.BlockSpec((512, _D), lambda i: (0, 0)),
        ],
        out_shape=[
            jax.ShapeDtypeStruct((_N, _D), jnp.float32),
            jax.ShapeDtypeStruct((_N, _D), jnp.float32),
            jax.ShapeDtypeStruct((_N, 1), jnp.float32),
            jax.ShapeDtypeStruct((512, _D), jnp.float32),
        ],
    )(x, wt, bvec, dps, b0, b1, b2)


def _main_body(ed_hbm, xg_hbm, dinv_hbm, c_hbm, parts_hbm,
               ebuf, cidv, lcv, dvb, xbuf, eebuf, acc_sh):
    c = lax.axis_index("c")
    s = lax.axis_index("s")
    base_node = c * _HALF

    def zb(i, _):
        r = i // 8
        k = i % 8
        xbuf[r, pl.ds(k * 16, 16)] = jnp.zeros((16,), jnp.float32)
        return 0
    lax.fori_loop(0, _CH * (_D // 16), zb, 0)

    def zc(j, _):
        pltpu.sync_copy(xbuf, acc_sh.at[pl.ds(s * _HPW + j * _CH, _CH)])
        return 0
    lax.fori_loop(0, _HPW // _CH, zc, 0)
    pltpu.sync_copy(xbuf.at[pl.ds(0, _HPW % _CH)],
                    acc_sh.at[pl.ds(s * _HPW + (_HPW // _CH) * _CH,
                                    _HPW % _CH)])
    plsc.subcore_barrier()

    def batch(g, _):
        gi = s * _CPS + g
        pltpu.sync_copy(ed_hbm.at[gi], ebuf)

        def cidb(j, _):
            sl = pl.ds(j * 16, 16)
            cidv[sl] = (ebuf[2, sl] * 8 + ebuf[3, sl]) * 8 + ebuf[4, sl]
            lc = ebuf[1, sl] - base_node
            ok = (lc >= 0) & (lc < _HALF)
            lcv[sl] = jnp.where(ok, lc, _HALF)
            return 0
        lax.fori_loop(0, _CH // 16, cidb, 0)

        pltpu.sync_copy(xg_hbm.at[ebuf.at[0]], xbuf)
        pltpu.sync_copy(dinv_hbm.at[ebuf.at[0]], dvb)
        pltpu.sync_copy(c_hbm.at[cidv], eebuf)

        @plsc.parallel_loop(0, _CH // 16)
        def _edge16(j):
            dv16 = dvb[pl.ds(j * 16, 16)]
            for lane in range(16):
                bi = j * 16 + lane
                dvsp = dv16.at[jnp.full((16,), lane, jnp.int32)].get(
                    mode="promise_in_bounds")
                for k in range(_D // 16):
                    sl = pl.ds(k * 16, 16)
                    xbuf[bi, sl] = jnp.maximum(
                        xbuf[bi, sl] + dvsp * eebuf[bi, sl], 0.0)

        pltpu.sync_copy(xbuf, acc_sh.at[lcv], add=True)
        return 0
    lax.fori_loop(0, _CPS, batch, 0)

    plsc.subcore_barrier()
    pltpu.sync_copy(acc_sh.at[pl.ds(s * _HPW, _HPW)],
                    parts_hbm.at[c, pl.ds(s * _HPW, _HPW)])


_main_call = functools.partial(
    pl.kernel,
    out_type=jax.ShapeDtypeStruct((_NC, _HP, _D), jnp.float32),
    mesh=_mesh,
    scratch_types=[
        pltpu.VMEM((5, _CH), jnp.int32),
        pltpu.VMEM((_CH,), jnp.int32),
        pltpu.VMEM((_CH,), jnp.int32),
        pltpu.VMEM((_CH,), jnp.float32),
        pltpu.VMEM((_CH, _D), jnp.float32),
        pltpu.VMEM((_CH, _D), jnp.float32),
        pltpu.VMEM_SHARED((_HP, _D), jnp.float32),
    ],
)(_main_body)


def _final_body(p_ref, xs_ref, root_ref, dinv_ref, out_ref):
    dinv = dinv_ref[...]
    selfv = jnp.maximum(xs_ref[...] + root_ref[...], 0.0) * (dinv * dinv)
    out_ref[...] = p_ref[...] * dinv + selfv


def _final_call(pfull, xs, root, dinv2):
    return pl.pallas_call(
        _final_body,
        grid=(_N // _R,),
        in_specs=[
            pl.BlockSpec((_R, _D), lambda i: (i, 0)),
            pl.BlockSpec((_R, _D), lambda i: (i, 0)),
            pl.BlockSpec((1, _D), lambda i: (0, 0)),
            pl.BlockSpec((_R, 1), lambda i: (i, 0)),
        ],
        out_specs=pl.BlockSpec((_R, _D), lambda i: (i, 0)),
        out_shape=jax.ShapeDtypeStruct((_N, _D), jnp.float32),
    )(pfull, xs, root, dinv2)


def kernel(x, edge_index, edge_attr, W, b, root_emb, B0, B1, B2):
    row = edge_index[0]
    col = edge_index[1]
    packed = jnp.stack(
        [row, col, edge_attr[:, 0], edge_attr[:, 1], edge_attr[:, 2]], 0)
    packed = packed.reshape(5, _NBALL, _CH).transpose(1, 0, 2)

    degp = _deg_call(packed)
    dps = (degp[0, :_N] + degp[1, :_N]).reshape(_N, 1)

    xs, xg, dinv2, C = _dense_call(
        x, W.T, b.reshape(1, _D), dps, B0, B1, B2)

    parts = _main_call(packed, xg, dinv2.reshape(_N), C)
    pfull = jnp.concatenate(
        [parts[0, :_HALF], parts[1, :_N - _HALF]], axis=0)

    return _final_call(pfull, xs, root_emb, dinv2)


# speedup vs baseline: 6.9827x; 1.2381x over previous
import functools

import jax
import jax.numpy as jnp
from jax import lax
from jax.experimental import pallas as pl
from jax.experimental.pallas import tpu as pltpu
from jax.experimental.pallas import tpu_sc as plsc

_N = 10000
_E = 320000
_D = 128
_NC = 2
_NS = 16
_NW = _NC * _NS
_CH = 80
_NBALL = _E // _CH
_CPW = _NBALL // _NW
_NP = 10240
_NPW = _NP // _NS
_R = 2000

_HALF = 5120
_HP = 5248
_HPW = _HP // _NS
_CPS = _NBALL // _NS

_mesh = plsc.VectorSubcoreMesh(
    core_axis_name="c", subcore_axis_name="s", num_cores=_NC, num_subcores=_NS)


def _deg_body(ed_hbm, out_hbm, ebuf, ones_v, zbuf, acc_sh):
    c = lax.axis_index("c")
    s = lax.axis_index("s")
    wid = c * _NS + s

    def zb(i, _):
        zbuf[pl.ds(i * 16, 16)] = jnp.zeros((16,), jnp.float32)
        return 0
    lax.fori_loop(0, _NPW // 16, zb, 0)

    def ob(i, _):
        ones_v[pl.ds(i * 16, 16)] = jnp.ones((16,), jnp.float32)
        return 0
    lax.fori_loop(0, _CH // 16, ob, 0)

    pltpu.sync_copy(zbuf, acc_sh.at[pl.ds(s * _NPW, _NPW)])
    plsc.subcore_barrier()

    def batch(g, _):
        gi = wid * _CPW + g
        pltpu.sync_copy(ed_hbm.at[gi, 0], ebuf)
        pltpu.sync_copy(ones_v, acc_sh.at[ebuf], add=True)
        return 0
    lax.fori_loop(0, _CPW, batch, 0)

    plsc.subcore_barrier()
    pltpu.sync_copy(acc_sh.at[pl.ds(s * _NPW, _NPW)],
                    out_hbm.at[c, pl.ds(s * _NPW, _NPW)])


_deg_call = functools.partial(
    pl.kernel,
    out_type=jax.ShapeDtypeStruct((_NC, _NP), jnp.float32),
    mesh=_mesh,
    scratch_types=[
        pltpu.VMEM((_CH,), jnp.int32),
        pltpu.VMEM((_CH,), jnp.float32),
        pltpu.VMEM((_NPW,), jnp.float32),
        pltpu.VMEM_SHARED((_NP,), jnp.float32),
    ],
)(_deg_body)


def _dense_body(x_ref, wt_ref, b_ref, dps_ref, b0_ref, b1_ref, b2_ref,
                xs_ref, xg_ref, dinv_ref, c_ref):
    xl = jnp.dot(x_ref[...], wt_ref[...], preferred_element_type=jnp.float32)
    xl = xl + b_ref[...]
    deg = dps_ref[...] + 1.0
    dinv = lax.rsqrt(deg)
    xs_ref[...] = xl
    xg_ref[...] = xl * dinv
    dinv_ref[...] = dinv
    t01 = (jnp.broadcast_to(b0_ref[...][:, None, :], (8, 8, _D))
           + b1_ref[...][None, :, :]).reshape(64, _D)
    c_ref[...] = (jnp.broadcast_to(t01[:, None, :], (64, 8, _D))
                  + b2_ref[...][None, :, :]).reshape(512, _D)


def _dense_call(x, wt, bvec, dps, b0, b1, b2):
    return pl.pallas_call(
        _dense_body,
        grid=(_N // _R,),
        in_specs=[
            pl.BlockSpec((_R, _D), lambda i: (i, 0)),
            pl.BlockSpec((_D, _D), lambda i: (0, 0)),
            pl.BlockSpec((1, _D), lambda i: (0, 0)),
            pl.BlockSpec((_R, 1), lambda i: (i, 0)),
            pl.BlockSpec((8, _D), lambda i: (0, 0)),
            pl.BlockSpec((8, _D), lambda i: (0, 0)),
            pl.BlockSpec((8, _D), lambda i: (0, 0)),
        ],
        out_specs=[
            pl.BlockSpec((_R, _D), lambda i: (i, 0)),
            pl.BlockSpec((_R, _D), lambda i: (i, 0)),
            pl.BlockSpec((_R, 1), lambda i: (i, 0)),
            pl.BlockSpec((512, _D), lambda i: (0, 0)),
        ],
        out_shape=[
            jax.ShapeDtypeStruct((_N, _D), jnp.float32),
            jax.ShapeDtypeStruct((_N, _D), jnp.float32),
            jax.ShapeDtypeStruct((_N, 1), jnp.float32),
            jax.ShapeDtypeStruct((512, _D), jnp.float32),
        ],
    )(x, wt, bvec, dps, b0, b1, b2)


def _main_body(ed_hbm, xg_hbm, dinv_hbm, c_hbm, parts_hbm,
               eb0, eb1, cv0, cv1, lv0, lv1, dv0, dv1, xb0, xb1, ee0, ee1,
               acc_sh, sed0, sed1, sgx0, sgx1, ssc0, ssc1):
    c = lax.axis_index("c")
    s = lax.axis_index("s")
    base_node = c * _HALF
    eb = (eb0, eb1)
    cv = (cv0, cv1)
    lv = (lv0, lv1)
    dv = (dv0, dv1)
    xb = (xb0, xb1)
    ee = (ee0, ee1)
    sed = (sed0, sed1)
    sgx = (sgx0, sgx1)
    ssc = (ssc0, ssc1)

    def zb(i, _):
        r = i // 8
        k = i % 8
        xb0[r, pl.ds(k * 16, 16)] = jnp.zeros((16,), jnp.float32)
        return 0
    lax.fori_loop(0, _CH * (_D // 16), zb, 0)

    def zc(j, _):
        pltpu.sync_copy(xb0, acc_sh.at[pl.ds(s * _HPW + j * _CH, _CH)])
        return 0
    lax.fori_loop(0, _HPW // _CH, zc, 0)
    pltpu.sync_copy(xb0.at[pl.ds(0, _HPW % _CH)],
                    acc_sh.at[pl.ds(s * _HPW + (_HPW // _CH) * _CH,
                                    _HPW % _CH)])
    plsc.subcore_barrier()

    def compute_idx(p):
        def cidb(j, _):
            sl = pl.ds(j * 16, 16)
            cv[p][sl] = (eb[p][2, sl] * 8 + eb[p][3, sl]) * 8 + eb[p][4, sl]
            lc = eb[p][1, sl] - base_node
            ok = (lc >= 0) & (lc < _HALF)
            lv[p][sl] = jnp.where(ok, lc, _HALF)
            return 0
        lax.fori_loop(0, _CH // 16, cidb, 0)

    def issue_gathers(p):
        pltpu.async_copy(xg_hbm.at[eb[p].at[0]], xb[p], sgx[p])
        pltpu.async_copy(dinv_hbm.at[eb[p].at[0]], dv[p], sgx[p])
        pltpu.async_copy(c_hbm.at[cv[p]], ee[p], sgx[p])

    def wait_gathers(p):
        pltpu.make_async_copy(xg_hbm.at[eb[p].at[0]], xb[p], sgx[p]).wait()
        pltpu.make_async_copy(dinv_hbm.at[eb[p].at[0]], dv[p], sgx[p]).wait()
        pltpu.make_async_copy(c_hbm.at[cv[p]], ee[p], sgx[p]).wait()

    def wait_scatter(p):
        pltpu.make_async_copy(xb[p], acc_sh.at[lv[p]], ssc[p]).wait()

    def compute_msgs(p):
        @plsc.parallel_loop(0, _CH // 16)
        def _edge16(j):
            dv16 = dv[p][pl.ds(j * 16, 16)]
            for lane in range(16):
                bi = j * 16 + lane
                dvsp = dv16.at[jnp.full((16,), lane, jnp.int32)].get(
                    mode="promise_in_bounds")
                for k in range(_D // 16):
                    sl = pl.ds(k * 16, 16)
                    xb[p][bi, sl] = jnp.maximum(
                        xb[p][bi, sl] + dvsp * ee[p][bi, sl], 0.0)

    pltpu.async_copy(ed_hbm.at[s * _CPS + 0], eb0, sed0)
    pltpu.async_copy(ed_hbm.at[s * _CPS + 1], eb1, sed1)
    pltpu.make_async_copy(ed_hbm.at[s * _CPS], eb0, sed0).wait()
    compute_idx(0)
    issue_gathers(0)

    def pipe(g2, _):
        for par in range(2):
            g = 2 * g2 + par
            p, q = par, 1 - par
            wait_gathers(p)

            @pl.when(g < _CPS - 2)
            def _():
                pltpu.async_copy(ed_hbm.at[s * _CPS + g + 2], eb[p], sed[p])

            compute_msgs(p)
            pltpu.async_copy(xb[p], acc_sh.at[lv[p]], ssc[p], add=True)

            @pl.when(g < _CPS - 1)
            def _():
                pltpu.make_async_copy(ed_hbm.at[s * _CPS], eb[q],
                                      sed[q]).wait()

                @pl.when(g >= 1)
                def _():
                    wait_scatter(q)

                compute_idx(q)
                issue_gathers(q)
        return 0
    lax.fori_loop(0, _CPS // 2, pipe, 0)

    wait_scatter(0)
    wait_scatter(1)

    plsc.subcore_barrier()
    pltpu.sync_copy(acc_sh.at[pl.ds(s * _HPW, _HPW)],
                    parts_hbm.at[c, pl.ds(s * _HPW, _HPW)])


_main_call = functools.partial(
    pl.kernel,
    out_type=jax.ShapeDtypeStruct((_NC, _HP, _D), jnp.float32),
    mesh=_mesh,
    scratch_types=[
        pltpu.VMEM((5, _CH), jnp.int32),
        pltpu.VMEM((5, _CH), jnp.int32),
        pltpu.VMEM((_CH,), jnp.int32),
        pltpu.VMEM((_CH,), jnp.int32),
        pltpu.VMEM((_CH,), jnp.int32),
        pltpu.VMEM((_CH,), jnp.int32),
        pltpu.VMEM((_CH,), jnp.float32),
        pltpu.VMEM((_CH,), jnp.float32),
        pltpu.VMEM((_CH, _D), jnp.float32),
        pltpu.VMEM((_CH, _D), jnp.float32),
        pltpu.VMEM((_CH, _D), jnp.float32),
        pltpu.VMEM((_CH, _D), jnp.float32),
        pltpu.VMEM_SHARED((_HP, _D), jnp.float32),
        pltpu.SemaphoreType.DMA,
        pltpu.SemaphoreType.DMA,
        pltpu.SemaphoreType.DMA,
        pltpu.SemaphoreType.DMA,
        pltpu.SemaphoreType.DMA,
        pltpu.SemaphoreType.DMA,
    ],
)(_main_body)


def _final_body(p_ref, xs_ref, root_ref, dinv_ref, out_ref):
    dinv = dinv_ref[...]
    selfv = jnp.maximum(xs_ref[...] + root_ref[...], 0.0) * (dinv * dinv)
    out_ref[...] = p_ref[...] * dinv + selfv


def _final_call(pfull, xs, root, dinv2):
    return pl.pallas_call(
        _final_body,
        grid=(_N // _R,),
        in_specs=[
            pl.BlockSpec((_R, _D), lambda i: (i, 0)),
            pl.BlockSpec((_R, _D), lambda i: (i, 0)),
            pl.BlockSpec((1, _D), lambda i: (0, 0)),
            pl.BlockSpec((_R, 1), lambda i: (i, 0)),
        ],
        out_specs=pl.BlockSpec((_R, _D), lambda i: (i, 0)),
        out_shape=jax.ShapeDtypeStruct((_N, _D), jnp.float32),
    )(pfull, xs, root, dinv2)


def kernel(x, edge_index, edge_attr, W, b, root_emb, B0, B1, B2):
    row = edge_index[0]
    col = edge_index[1]
    packed = jnp.stack(
        [row, col, edge_attr[:, 0], edge_attr[:, 1], edge_attr[:, 2]], 0)
    packed = packed.reshape(5, _NBALL, _CH).transpose(1, 0, 2)

    degp = _deg_call(packed)
    dps = (degp[0, :_N] + degp[1, :_N]).reshape(_N, 1)

    xs, xg, dinv2, C = _dense_call(
        x, W.T, b.reshape(1, _D), dps, B0, B1, B2)

    parts = _main_call(packed, xg, dinv2.reshape(_N), C)
    pfull = jnp.concatenate(
        [parts[0, :_HALF], parts[1, :_N - _HALF]], axis=0)

    return _final_call(pfull, xs, root_emb, dinv2)


# speedup vs baseline: 8.6024x; 1.2320x over previous
import functools

import jax
import jax.numpy as jnp
from jax import lax
from jax.experimental import pallas as pl
from jax.experimental.pallas import tpu as pltpu
from jax.experimental.pallas import tpu_sc as plsc

_N = 10000
_E = 320000
_D = 128
_NC = 2
_NS = 16
_NW = _NC * _NS
_CH = 80
_NBALL = _E // _CH
_CPW = _NBALL // _NW
_NP = 10240
_NPW = _NP // _NS
_R = 2000

_HALF = 5120
_HP = 5248
_HPW = _HP // _NS
_CPS = _NBALL // _NS

_mesh = plsc.VectorSubcoreMesh(
    core_axis_name="c", subcore_axis_name="s", num_cores=_NC, num_subcores=_NS)


def _deg_body(ed_hbm, out_hbm, ebuf, ones_v, zbuf, acc_sh):
    c = lax.axis_index("c")
    s = lax.axis_index("s")
    wid = c * _NS + s

    def zb(i, _):
        zbuf[pl.ds(i * 16, 16)] = jnp.zeros((16,), jnp.float32)
        return 0
    lax.fori_loop(0, _NPW // 16, zb, 0)

    def ob(i, _):
        ones_v[pl.ds(i * 16, 16)] = jnp.ones((16,), jnp.float32)
        return 0
    lax.fori_loop(0, _CH // 16, ob, 0)

    pltpu.sync_copy(zbuf, acc_sh.at[pl.ds(s * _NPW, _NPW)])
    plsc.subcore_barrier()

    def batch(g, _):
        gi = wid * _CPW + g
        pltpu.sync_copy(ed_hbm.at[gi, 0], ebuf)
        pltpu.sync_copy(ones_v, acc_sh.at[ebuf], add=True)
        return 0
    lax.fori_loop(0, _CPW, batch, 0)

    plsc.subcore_barrier()
    pltpu.sync_copy(acc_sh.at[pl.ds(s * _NPW, _NPW)],
                    out_hbm.at[c, pl.ds(s * _NPW, _NPW)])


_deg_call = functools.partial(
    pl.kernel,
    out_type=jax.ShapeDtypeStruct((_NC, _NP), jnp.float32),
    mesh=_mesh,
    scratch_types=[
        pltpu.VMEM((_CH,), jnp.int32),
        pltpu.VMEM((_CH,), jnp.float32),
        pltpu.VMEM((_NPW,), jnp.float32),
        pltpu.VMEM_SHARED((_NP,), jnp.float32),
    ],
)(_deg_body)


def _dense_body(x_ref, wt_ref, b_ref, dps_ref, b0_ref, b1_ref, b2_ref,
                xs_ref, xg_ref, dinv_ref, c_ref, dinvw_ref):
    xl = jnp.dot(x_ref[...], wt_ref[...], preferred_element_type=jnp.float32)
    xl = xl + b_ref[...]
    deg = dps_ref[...] + 1.0
    dinv = lax.rsqrt(deg)
    xs_ref[...] = xl
    xg_ref[...] = xl * dinv
    dinv_ref[...] = dinv
    dinvw_ref[...] = jnp.broadcast_to(dinv, xl.shape)
    t01 = (jnp.broadcast_to(b0_ref[...][:, None, :], (8, 8, _D))
           + b1_ref[...][None, :, :]).reshape(64, _D)
    c_ref[...] = (jnp.broadcast_to(t01[:, None, :], (64, 8, _D))
                  + b2_ref[...][None, :, :]).reshape(512, _D)


def _dense_call(x, wt, bvec, dps, b0, b1, b2):
    return pl.pallas_call(
        _dense_body,
        grid=(_N // _R,),
        in_specs=[
            pl.BlockSpec((_R, _D), lambda i: (i, 0)),
            pl.BlockSpec((_D, _D), lambda i: (0, 0)),
            pl.BlockSpec((1, _D), lambda i: (0, 0)),
            pl.BlockSpec((_R, 1), lambda i: (i, 0)),
            pl.BlockSpec((8, _D), lambda i: (0, 0)),
            pl.BlockSpec((8, _D), lambda i: (0, 0)),
            pl.BlockSpec((8, _D), lambda i: (0, 0)),
        ],
        out_specs=[
            pl.BlockSpec((_R, _D), lambda i: (i, 0)),
            pl.BlockSpec((_R, _D), lambda i: (i, 0)),
            pl.BlockSpec((_R, 1), lambda i: (i, 0)),
            pl.BlockSpec((512, _D), lambda i: (0, 0)),
            pl.BlockSpec((_R, _D), lambda i: (i, 0)),
        ],
        out_shape=[
            jax.ShapeDtypeStruct((_N, _D), jnp.float32),
            jax.ShapeDtypeStruct((_N, _D), jnp.float32),
            jax.ShapeDtypeStruct((_N, 1), jnp.float32),
            jax.ShapeDtypeStruct((512, _D), jnp.float32),
            jax.ShapeDtypeStruct((_N, _D), jnp.float32),
        ],
    )(x, wt, bvec, dps, b0, b1, b2)


def _main_body(ed_hbm, xg_hbm, dinvw_hbm, c_hbm, parts_hbm,
               eb0, eb1, cv0, cv1, lv0, lv1, dv0, dv1, xb0, xb1, ee0, ee1,
               acc_sh, sed0, sed1, sgx0, sgx1, ssc0, ssc1):
    c = lax.axis_index("c")
    s = lax.axis_index("s")
    base_node = c * _HALF
    eb = (eb0, eb1)
    cv = (cv0, cv1)
    lv = (lv0, lv1)
    dv = (dv0, dv1)
    xb = (xb0, xb1)
    ee = (ee0, ee1)
    sed = (sed0, sed1)
    sgx = (sgx0, sgx1)
    ssc = (ssc0, ssc1)

    def zb(i, _):
        r = i // 8
        k = i % 8
        xb0[r, pl.ds(k * 16, 16)] = jnp.zeros((16,), jnp.float32)
        return 0
    lax.fori_loop(0, _CH * (_D // 16), zb, 0)

    def zc(j, _):
        pltpu.sync_copy(xb0, acc_sh.at[pl.ds(s * _HPW + j * _CH, _CH)])
        return 0
    lax.fori_loop(0, _HPW // _CH, zc, 0)
    pltpu.sync_copy(xb0.at[pl.ds(0, _HPW % _CH)],
                    acc_sh.at[pl.ds(s * _HPW + (_HPW // _CH) * _CH,
                                    _HPW % _CH)])
    plsc.subcore_barrier()

    def compute_idx(p):
        def cidb(j, _):
            sl = pl.ds(j * 16, 16)
            cv[p][sl] = (eb[p][2, sl] * 8 + eb[p][3, sl]) * 8 + eb[p][4, sl]
            lc = eb[p][1, sl] - base_node
            ok = (lc >= 0) & (lc < _HALF)
            lv[p][sl] = jnp.where(ok, lc, _HALF)
            return 0
        lax.fori_loop(0, _CH // 16, cidb, 0)

    def issue_gathers(p):
        pltpu.async_copy(xg_hbm.at[eb[p].at[0]], xb[p], sgx[p])
        pltpu.async_copy(dinvw_hbm.at[eb[p].at[0]], dv[p], sgx[p])
        pltpu.async_copy(c_hbm.at[cv[p]], ee[p], sgx[p])

    def wait_gathers(p):
        pltpu.make_async_copy(xg_hbm.at[eb[p].at[0]], xb[p], sgx[p]).wait()
        pltpu.make_async_copy(dinvw_hbm.at[eb[p].at[0]], dv[p],
                              sgx[p]).wait()
        pltpu.make_async_copy(c_hbm.at[cv[p]], ee[p], sgx[p]).wait()

    def wait_scatter(p):
        pltpu.make_async_copy(xb[p], acc_sh.at[lv[p]], ssc[p]).wait()

    def compute_msgs(p):
        @plsc.parallel_loop(0, _CH)
        def _row(r):
            for k in range(_D // 16):
                sl = pl.ds(k * 16, 16)
                xb[p][r, sl] = jnp.maximum(
                    xb[p][r, sl] + dv[p][r, sl] * ee[p][r, sl], 0.0)

    pltpu.async_copy(ed_hbm.at[s * _CPS + 0], eb0, sed0)
    pltpu.async_copy(ed_hbm.at[s * _CPS + 1], eb1, sed1)
    pltpu.make_async_copy(ed_hbm.at[s * _CPS], eb0, sed0).wait()
    compute_idx(0)
    issue_gathers(0)

    def pipe(g2, _):
        for par in range(2):
            g = 2 * g2 + par
            p, q = par, 1 - par
            wait_gathers(p)

            @pl.when(g < _CPS - 2)
            def _():
                pltpu.async_copy(ed_hbm.at[s * _CPS + g + 2], eb[p], sed[p])

            compute_msgs(p)
            pltpu.async_copy(xb[p], acc_sh.at[lv[p]], ssc[p], add=True)

            @pl.when(g < _CPS - 1)
            def _():
                pltpu.make_async_copy(ed_hbm.at[s * _CPS], eb[q],
                                      sed[q]).wait()

                @pl.when(g >= 1)
                def _():
                    wait_scatter(q)

                compute_idx(q)
                issue_gathers(q)
        return 0
    lax.fori_loop(0, _CPS // 2, pipe, 0)

    wait_scatter(0)
    wait_scatter(1)

    plsc.subcore_barrier()
    pltpu.sync_copy(acc_sh.at[pl.ds(s * _HPW, _HPW)],
                    parts_hbm.at[c, pl.ds(s * _HPW, _HPW)])


_main_call = functools.partial(
    pl.kernel,
    out_type=jax.ShapeDtypeStruct((_NC, _HP, _D), jnp.float32),
    mesh=_mesh,
    scratch_types=[
        pltpu.VMEM((5, _CH), jnp.int32),
        pltpu.VMEM((5, _CH), jnp.int32),
        pltpu.VMEM((_CH,), jnp.int32),
        pltpu.VMEM((_CH,), jnp.int32),
        pltpu.VMEM((_CH,), jnp.int32),
        pltpu.VMEM((_CH,), jnp.int32),
        pltpu.VMEM((_CH, _D), jnp.float32),
        pltpu.VMEM((_CH, _D), jnp.float32),
        pltpu.VMEM((_CH, _D), jnp.float32),
        pltpu.VMEM((_CH, _D), jnp.float32),
        pltpu.VMEM((_CH, _D), jnp.float32),
        pltpu.VMEM((_CH, _D), jnp.float32),
        pltpu.VMEM_SHARED((_HP, _D), jnp.float32),
        pltpu.SemaphoreType.DMA,
        pltpu.SemaphoreType.DMA,
        pltpu.SemaphoreType.DMA,
        pltpu.SemaphoreType.DMA,
        pltpu.SemaphoreType.DMA,
        pltpu.SemaphoreType.DMA,
    ],
)(_main_body)


def _final_body(p_ref, xs_ref, root_ref, dinv_ref, out_ref):
    dinv = dinv_ref[...]
    selfv = jnp.maximum(xs_ref[...] + root_ref[...], 0.0) * (dinv * dinv)
    out_ref[...] = p_ref[...] * dinv + selfv


def _final_call(pfull, xs, root, dinv2):
    return pl.pallas_call(
        _final_body,
        grid=(_N // _R,),
        in_specs=[
            pl.BlockSpec((_R, _D), lambda i: (i, 0)),
            pl.BlockSpec((_R, _D), lambda i: (i, 0)),
            pl.BlockSpec((1, _D), lambda i: (0, 0)),
            pl.BlockSpec((_R, 1), lambda i: (i, 0)),
        ],
        out_specs=pl.BlockSpec((_R, _D), lambda i: (i, 0)),
        out_shape=jax.ShapeDtypeStruct((_N, _D), jnp.float32),
    )(pfull, xs, root, dinv2)


def kernel(x, edge_index, edge_attr, W, b, root_emb, B0, B1, B2):
    row = edge_index[0]
    col = edge_index[1]
    packed = jnp.stack(
        [row, col, edge_attr[:, 0], edge_attr[:, 1], edge_attr[:, 2]], 0)
    packed = packed.reshape(5, _NBALL, _CH).transpose(1, 0, 2)

    degp = _deg_call(packed)
    dps = (degp[0, :_N] + degp[1, :_N]).reshape(_N, 1)

    xs, xg, dinv2, C, dinvw = _dense_call(
        x, W.T, b.reshape(1, _D), dps, B0, B1, B2)

    parts = _main_call(packed, xg, dinvw, C)
    pfull = jnp.concatenate(
        [parts[0, :_HALF], parts[1, :_N - _HALF]], axis=0)

    return _final_call(pfull, xs, root_emb, dinv2)


# speedup vs baseline: 8.9098x; 1.0357x over previous
import functools

import jax
import jax.numpy as jnp
from jax import lax
from jax.experimental import pallas as pl
from jax.experimental.pallas import tpu as pltpu
from jax.experimental.pallas import tpu_sc as plsc

_N = 10000
_E = 320000
_D = 128
_NC = 2
_NS = 16
_NW = _NC * _NS
_CH = 80
_NBK = _E // _CH
_R = 2000

_NP = 10240
_NPW1 = _NP // _NS
_K1B = _NBK // _NW
_K1X = _NBK - _K1B * _NW

_HALF = 5120
_HP = 5248
_HPW = _HP // _NS
_K3B = _NBK // _NS
_K3X = _NBK - _K3B * _NS

_mesh = plsc.VectorSubcoreMesh(
    core_axis_name="c", subcore_axis_name="s", num_cores=_NC, num_subcores=_NS)


def _deg_body(ed_hbm, out_hbm, ebuf, ones_v, zbuf, acc_sh):
    c = lax.axis_index("c")
    s = lax.axis_index("s")
    wid = c * _NS + s
    start = wid * _K1B + jnp.minimum(wid, _K1X)
    cnt = _K1B + jnp.where(wid < _K1X, 1, 0)

    def zb(i, _):
        zbuf[pl.ds(i * 16, 16)] = jnp.zeros((16,), jnp.float32)
        return 0
    lax.fori_loop(0, _NPW1 // 16, zb, 0)

    def ob(i, _):
        ones_v[pl.ds(i * 16, 16)] = jnp.ones((16,), jnp.float32)
        return 0
    lax.fori_loop(0, _CH // 16, ob, 0)

    pltpu.sync_copy(zbuf, acc_sh.at[pl.ds(s * _NPW1, _NPW1)])
    plsc.subcore_barrier()

    def batch(g, _):
        pltpu.sync_copy(ed_hbm.at[start + g, 0], ebuf)
        pltpu.sync_copy(ones_v, acc_sh.at[ebuf], add=True)
        return 0
    lax.fori_loop(0, cnt, batch, 0)

    plsc.subcore_barrier()
    pltpu.sync_copy(acc_sh.at[pl.ds(s * _NPW1, _NPW1)],
                    out_hbm.at[c, pl.ds(s * _NPW1, _NPW1)])


_deg_call = functools.partial(
    pl.kernel,
    out_type=jax.ShapeDtypeStruct((_NC, _NP), jnp.float32),
    mesh=_mesh,
    scratch_types=[
        pltpu.VMEM((_CH,), jnp.int32),
        pltpu.VMEM((_CH,), jnp.float32),
        pltpu.VMEM((_NPW1,), jnp.float32),
        pltpu.VMEM_SHARED((_NP,), jnp.float32),
    ],
)(_deg_body)


def _dense_body(x_ref, wt_ref, b_ref, dps_ref, b0_ref, b1_ref, b2_ref,
                xs_ref, dinv_ref, c_ref, xgd_ref):
    xl = jnp.dot(x_ref[...], wt_ref[...], preferred_element_type=jnp.float32)
    xl = xl + b_ref[...]
    deg = dps_ref[...] + 1.0
    dinv = lax.rsqrt(deg)
    xs_ref[...] = xl
    dinv_ref[...] = dinv
    xgd_ref[:, :_D] = xl * dinv
    xgd_ref[:, _D:] = jnp.broadcast_to(dinv, xl.shape)
    t01 = (jnp.broadcast_to(b0_ref[...][:, None, :], (8, 8, _D))
           + b1_ref[...][None, :, :]).reshape(64, _D)
    c_ref[...] = (jnp.broadcast_to(t01[:, None, :], (64, 8, _D))
                  + b2_ref[...][None, :, :]).reshape(512, _D)


def _dense_call(x, wt, bvec, dps, b0, b1, b2):
    return pl.pallas_call(
        _dense_body,
        grid=(_N // _R,),
        in_specs=[
            pl.BlockSpec((_R, _D), lambda i: (i, 0)),
            pl.BlockSpec((_D, _D), lambda i: (0, 0)),
            pl.BlockSpec((1, _D), lambda i: (0, 0)),
            pl.BlockSpec((_R, 1), lambda i: (i, 0)),
            pl.BlockSpec((8, _D), lambda i: (0, 0)),
            pl.BlockSpec((8, _D), lambda i: (0, 0)),
            pl.BlockSpec((8, _D), lambda i: (0, 0)),
        ],
        out_specs=[
            pl.BlockSpec((_R, _D), lambda i: (i, 0)),
            pl.BlockSpec((_R, 1), lambda i: (i, 0)),
            pl.BlockSpec((512, _D), lambda i: (0, 0)),
            pl.BlockSpec((_R, 2 * _D), lambda i: (i, 0)),
        ],
        out_shape=[
            jax.ShapeDtypeStruct((_N, _D), jnp.float32),
            jax.ShapeDtypeStruct((_N, 1), jnp.float32),
            jax.ShapeDtypeStruct((512, _D), jnp.float32),
            jax.ShapeDtypeStruct((_N, 2 * _D), jnp.float32),
        ],
    )(x, wt, bvec, dps, b0, b1, b2)


def _main_body(ed_hbm, xgd_hbm, c_hbm, parts_hbm,
               eb0, eb1, cv0, cv1, lv0, lv1, xg0, xg1, ee0, ee1,
               acc_sh, sed0, sed1, sgx0, sgx1, ssc0, ssc1):
    c = lax.axis_index("c")
    s = lax.axis_index("s")
    base_node = c * _HALF
    start = s * _K3B + jnp.minimum(s, _K3X)
    cnt = _K3B + jnp.where(s < _K3X, 1, 0)
    eb = (eb0, eb1)
    cv = (cv0, cv1)
    lv = (lv0, lv1)
    xg = (xg0, xg1)
    ee = (ee0, ee1)
    sed = (sed0, sed1)
    sgx = (sgx0, sgx1)
    ssc = (ssc0, ssc1)

    def zb(i, _):
        r = i // 8
        k = i % 8
        ee0[r, pl.ds(k * 16, 16)] = jnp.zeros((16,), jnp.float32)
        return 0
    lax.fori_loop(0, _CH * (_D // 16), zb, 0)

    def zc(j, _):
        pltpu.sync_copy(ee0, acc_sh.at[pl.ds(s * _HPW + j * _CH, _CH)])
        return 0
    lax.fori_loop(0, _HPW // _CH, zc, 0)
    pltpu.sync_copy(ee0.at[pl.ds(0, _HPW % _CH)],
                    acc_sh.at[pl.ds(s * _HPW + (_HPW // _CH) * _CH,
                                    _HPW % _CH)])
    plsc.subcore_barrier()

    def compute_idx(p):
        def cidb(j, _):
            sl = pl.ds(j * 16, 16)
            cv[p][sl] = (eb[p][2, sl] * 8 + eb[p][3, sl]) * 8 + eb[p][4, sl]
            lc = eb[p][1, sl] - base_node
            ok = (lc >= 0) & (lc < _HALF)
            lv[p][sl] = jnp.where(ok, lc, _HALF)
            return 0
        lax.fori_loop(0, _CH // 16, cidb, 0)

    def issue_gathers(p):
        pltpu.async_copy(xgd_hbm.at[eb[p].at[0]], xg[p], sgx[p])
        pltpu.async_copy(c_hbm.at[cv[p]], ee[p], sgx[p])

    def wait_gathers(p):
        pltpu.make_async_copy(xgd_hbm.at[eb[p].at[0]], xg[p], sgx[p]).wait()
        pltpu.make_async_copy(c_hbm.at[cv[p]], ee[p], sgx[p]).wait()

    def wait_scatter(p):
        pltpu.make_async_copy(ee[p], acc_sh.at[lv[p]], ssc[p]).wait()

    def compute_msgs(p):
        @plsc.parallel_loop(0, _CH)
        def _row(r):
            for k in range(_D // 16):
                sl = pl.ds(k * 16, 16)
                slh = pl.ds(_D + k * 16, 16)
                ee[p][r, sl] = jnp.maximum(
                    xg[p][r, sl] + xg[p][r, slh] * ee[p][r, sl], 0.0)

    pltpu.async_copy(ed_hbm.at[start], eb0, sed0)
    pltpu.async_copy(ed_hbm.at[start + 1], eb1, sed1)
    pltpu.make_async_copy(ed_hbm.at[start], eb0, sed0).wait()
    compute_idx(0)
    issue_gathers(0)

    def pipe(g2, _):
        for par in range(2):
            g = 2 * g2 + par
            p, q = par, 1 - par

            @pl.when(g < cnt)
            def _():
                wait_gathers(p)

                @pl.when(g < cnt - 2)
                def _():
                    pltpu.async_copy(ed_hbm.at[start + g + 2], eb[p], sed[p])

                compute_msgs(p)
                pltpu.async_copy(ee[p], acc_sh.at[lv[p]], ssc[p], add=True)

                @pl.when(g < cnt - 1)
                def _():
                    pltpu.make_async_copy(ed_hbm.at[start], eb[q],
                                          sed[q]).wait()

                    @pl.when(g >= 1)
                    def _():
                        wait_scatter(q)

                    compute_idx(q)
                    issue_gathers(q)
        return 0
    lax.fori_loop(0, (_K3B + _K3X + 1) // 2, pipe, 0)

    wait_scatter(0)
    wait_scatter(1)

    plsc.subcore_barrier()
    pltpu.sync_copy(acc_sh.at[pl.ds(s * _HPW, _HPW)],
                    parts_hbm.at[c, pl.ds(s * _HPW, _HPW)])


_main_call = functools.partial(
    pl.kernel,
    out_type=jax.ShapeDtypeStruct((_NC, _HP, _D), jnp.float32),
    mesh=_mesh,
    scratch_types=[
        pltpu.VMEM((5, _CH), jnp.int32),
        pltpu.VMEM((5, _CH), jnp.int32),
        pltpu.VMEM((_CH,), jnp.int32),
        pltpu.VMEM((_CH,), jnp.int32),
        pltpu.VMEM((_CH,), jnp.int32),
        pltpu.VMEM((_CH,), jnp.int32),
        pltpu.VMEM((_CH, 2 * _D), jnp.float32),
        pltpu.VMEM((_CH, 2 * _D), jnp.float32),
        pltpu.VMEM((_CH, _D), jnp.float32),
        pltpu.VMEM((_CH, _D), jnp.float32),
        pltpu.VMEM_SHARED((_HP, _D), jnp.float32),
        pltpu.SemaphoreType.DMA,
        pltpu.SemaphoreType.DMA,
        pltpu.SemaphoreType.DMA,
        pltpu.SemaphoreType.DMA,
        pltpu.SemaphoreType.DMA,
        pltpu.SemaphoreType.DMA,
    ],
)(_main_body)


def _final_body(p_ref, xs_ref, root_ref, dinv_ref, out_ref):
    dinv = dinv_ref[...]
    selfv = jnp.maximum(xs_ref[...] + root_ref[...], 0.0) * (dinv * dinv)
    out_ref[...] = p_ref[...] * dinv + selfv


def _final_call(pfull, xs, root, dinv2):
    return pl.pallas_call(
        _final_body,
        grid=(_N // _R,),
        in_specs=[
            pl.BlockSpec((_R, _D), lambda i: (i, 0)),
            pl.BlockSpec((_R, _D), lambda i: (i, 0)),
            pl.BlockSpec((1, _D), lambda i: (0, 0)),
            pl.BlockSpec((_R, 1), lambda i: (i, 0)),
        ],
        out_specs=pl.BlockSpec((_R, _D), lambda i: (i, 0)),
        out_shape=jax.ShapeDtypeStruct((_N, _D), jnp.float32),
    )(pfull, xs, root, dinv2)


def kernel(x, edge_index, edge_attr, W, b, root_emb, B0, B1, B2):
    row = edge_index[0]
    col = edge_index[1]
    packed = jnp.stack(
        [row, col, edge_attr[:, 0], edge_attr[:, 1], edge_attr[:, 2]], 0)
    packed = packed.reshape(5, _NBK, _CH).transpose(1, 0, 2)

    degp = _deg_call(packed)
    dps = (degp[0, :_N] + degp[1, :_N]).reshape(_N, 1)

    xs, dinv2, C, xgd = _dense_call(
        x, W.T, b.reshape(1, _D), dps, B0, B1, B2)

    parts = _main_call(packed, xgd, C)
    pfull = jnp.concatenate(
        [parts[0, :_HALF], parts[1, :_N - _HALF]], axis=0)

    return _final_call(pfull, xs, root_emb, dinv2)


# speedup vs baseline: 9.3858x; 1.0534x over previous
import functools

import jax
import jax.numpy as jnp
from jax import lax
from jax.experimental import pallas as pl
from jax.experimental.pallas import tpu as pltpu
from jax.experimental.pallas import tpu_sc as plsc

_N = 10000
_E = 320000
_D = 128
_NC = 2
_NS = 16
_NW = _NC * _NS
_CH = 80
_GW = 256
_NBK = _E // _CH
_R = 2000

_NP = 10240
_NPW1 = _NP // _NS
_K1B = _NBK // _NW
_K1X = _NBK - _K1B * _NW

_HALF = 5120
_HP = 5248
_HPW = _HP // _NS
_K3B = _NBK // _NS
_K3X = _NBK - _K3B * _NS

_mesh = plsc.VectorSubcoreMesh(
    core_axis_name="c", subcore_axis_name="s", num_cores=_NC, num_subcores=_NS)


def _deg_body(ed_hbm, out_hbm, ebuf, ones_v, zbuf, acc_sh):
    c = lax.axis_index("c")
    s = lax.axis_index("s")
    wid = c * _NS + s
    start = wid * _K1B + jnp.minimum(wid, _K1X)
    cnt = _K1B + jnp.where(wid < _K1X, 1, 0)

    def zb(i, _):
        zbuf[pl.ds(i * 16, 16)] = jnp.zeros((16,), jnp.float32)
        return 0
    lax.fori_loop(0, _NPW1 // 16, zb, 0)

    def ob(i, _):
        ones_v[pl.ds(i * 16, 16)] = jnp.ones((16,), jnp.float32)
        return 0
    lax.fori_loop(0, _CH // 16, ob, 0)

    pltpu.sync_copy(zbuf, acc_sh.at[pl.ds(s * _NPW1, _NPW1)])
    plsc.subcore_barrier()

    def batch(g, _):
        pltpu.sync_copy(ed_hbm.at[start + g, 0], ebuf)
        pltpu.sync_copy(ones_v, acc_sh.at[ebuf], add=True)
        return 0
    lax.fori_loop(0, cnt, batch, 0)

    plsc.subcore_barrier()
    pltpu.sync_copy(acc_sh.at[pl.ds(s * _NPW1, _NPW1)],
                    out_hbm.at[c, pl.ds(s * _NPW1, _NPW1)])


_deg_call = functools.partial(
    pl.kernel,
    out_type=jax.ShapeDtypeStruct((_NC, _NP), jnp.float32),
    mesh=_mesh,
    scratch_types=[
        pltpu.VMEM((_CH,), jnp.int32),
        pltpu.VMEM((_CH,), jnp.float32),
        pltpu.VMEM((_NPW1,), jnp.float32),
        pltpu.VMEM_SHARED((_NP,), jnp.float32),
    ],
)(_deg_body)


def _dense_body(x_ref, wt_ref, b_ref, dps_ref, b0_ref, b1_ref, b2_ref,
                xs_ref, dinv_ref, c_ref, xgd_ref):
    xl = jnp.dot(x_ref[...], wt_ref[...], preferred_element_type=jnp.float32)
    xl = xl + b_ref[...]
    deg = dps_ref[...] + 1.0
    dinv = lax.rsqrt(deg)
    xs_ref[...] = xl
    dinv_ref[...] = dinv
    xgd_ref[:, :_D] = xl * dinv
    xgd_ref[:, _D:] = jnp.broadcast_to(dinv, (xl.shape[0], _GW - _D))
    t01 = (jnp.broadcast_to(b0_ref[...][:, None, :], (8, 8, _D))
           + b1_ref[...][None, :, :]).reshape(64, _D)
    c_ref[...] = (jnp.broadcast_to(t01[:, None, :], (64, 8, _D))
                  + b2_ref[...][None, :, :]).reshape(512, _D)


def _dense_call(x, wt, bvec, dps, b0, b1, b2):
    return pl.pallas_call(
        _dense_body,
        grid=(_N // _R,),
        in_specs=[
            pl.BlockSpec((_R, _D), lambda i: (i, 0)),
            pl.BlockSpec((_D, _D), lambda i: (0, 0)),
            pl.BlockSpec((1, _D), lambda i: (0, 0)),
            pl.BlockSpec((_R, 1), lambda i: (i, 0)),
            pl.BlockSpec((8, _D), lambda i: (0, 0)),
            pl.BlockSpec((8, _D), lambda i: (0, 0)),
            pl.BlockSpec((8, _D), lambda i: (0, 0)),
        ],
        out_specs=[
            pl.BlockSpec((_R, _D), lambda i: (i, 0)),
            pl.BlockSpec((_R, 1), lambda i: (i, 0)),
            pl.BlockSpec((512, _D), lambda i: (0, 0)),
            pl.BlockSpec((_R, _GW), lambda i: (i, 0)),
        ],
        out_shape=[
            jax.ShapeDtypeStruct((_N, _D), jnp.float32),
            jax.ShapeDtypeStruct((_N, 1), jnp.float32),
            jax.ShapeDtypeStruct((512, _D), jnp.float32),
            jax.ShapeDtypeStruct((_N, _GW), jnp.float32),
        ],
    )(x, wt, bvec, dps, b0, b1, b2)


def _main_body(ed_hbm, xgd_hbm, c_hbm, parts_hbm,
               eb0, eb1, cv0, cv1, lv0, lv1, xg0, xg1, ee0, ee1,
               acc_sh, sed0, sed1, sgx0, sgx1, ssc0, ssc1):
    c = lax.axis_index("c")
    s = lax.axis_index("s")
    base_node = c * _HALF
    start = s * _K3B + jnp.minimum(s, _K3X)
    cnt = _K3B + jnp.where(s < _K3X, 1, 0)
    eb = (eb0, eb1)
    cv = (cv0, cv1)
    lv = (lv0, lv1)
    xg = (xg0, xg1)
    ee = (ee0, ee1)
    sed = (sed0, sed1)
    sgx = (sgx0, sgx1)
    ssc = (ssc0, ssc1)

    def zb(i, _):
        r = i // 8
        k = i % 8
        ee0[r, pl.ds(k * 16, 16)] = jnp.zeros((16,), jnp.float32)
        return 0
    lax.fori_loop(0, _CH * (_D // 16), zb, 0)

    def zc(j, _):
        pltpu.sync_copy(ee0, acc_sh.at[pl.ds(s * _HPW + j * _CH, _CH)])
        return 0
    lax.fori_loop(0, _HPW // _CH, zc, 0)
    pltpu.sync_copy(ee0.at[pl.ds(0, _HPW % _CH)],
                    acc_sh.at[pl.ds(s * _HPW + (_HPW // _CH) * _CH,
                                    _HPW % _CH)])
    plsc.subcore_barrier()

    def compute_idx(p):
        def cidb(j, _):
            sl = pl.ds(j * 16, 16)
            cv[p][sl] = (eb[p][2, sl] * 8 + eb[p][3, sl]) * 8 + eb[p][4, sl]
            lc = eb[p][1, sl] - base_node
            ok = (lc >= 0) & (lc < _HALF)
            lv[p][sl] = jnp.where(ok, lc, _HALF)
            return 0
        lax.fori_loop(0, _CH // 16, cidb, 0)

    def issue_gathers(p):
        pltpu.async_copy(xgd_hbm.at[eb[p].at[0]], xg[p], sgx[p])
        pltpu.async_copy(c_hbm.at[cv[p]], ee[p], sgx[p])

    def wait_gathers(p):
        pltpu.make_async_copy(xgd_hbm.at[eb[p].at[0]], xg[p], sgx[p]).wait()
        pltpu.make_async_copy(c_hbm.at[cv[p]], ee[p], sgx[p]).wait()

    def wait_scatter(p):
        pltpu.make_async_copy(ee[p], acc_sh.at[lv[p]], ssc[p]).wait()

    def compute_msgs(p):
        @plsc.parallel_loop(0, _CH)
        def _row(r):
            dvsp = xg[p][r, pl.ds(_D, 16)]
            for k in range(_D // 16):
                sl = pl.ds(k * 16, 16)
                ee[p][r, sl] = jnp.maximum(
                    xg[p][r, sl] + dvsp * ee[p][r, sl], 0.0)

    pltpu.async_copy(ed_hbm.at[start], eb0, sed0)
    pltpu.async_copy(ed_hbm.at[start + 1], eb1, sed1)
    pltpu.make_async_copy(ed_hbm.at[start], eb0, sed0).wait()
    compute_idx(0)
    issue_gathers(0)

    def pipe(g2, _):
        for par in range(2):
            g = 2 * g2 + par
            p, q = par, 1 - par

            @pl.when(g < cnt)
            def _():
                wait_gathers(p)

                @pl.when(g < cnt - 2)
                def _():
                    pltpu.async_copy(ed_hbm.at[start + g + 2], eb[p], sed[p])

                compute_msgs(p)
                pltpu.async_copy(ee[p], acc_sh.at[lv[p]], ssc[p], add=True)

                @pl.when(g < cnt - 1)
                def _():
                    pltpu.make_async_copy(ed_hbm.at[start], eb[q],
                                          sed[q]).wait()

                    @pl.when(g >= 1)
                    def _():
                        wait_scatter(q)

                    compute_idx(q)
                    issue_gathers(q)
        return 0
    lax.fori_loop(0, (_K3B + _K3X + 1) // 2, pipe, 0)

    wait_scatter(0)
    wait_scatter(1)

    plsc.subcore_barrier()
    pltpu.sync_copy(acc_sh.at[pl.ds(s * _HPW, _HPW)],
                    parts_hbm.at[c, pl.ds(s * _HPW, _HPW)])


_main_call = functools.partial(
    pl.kernel,
    out_type=jax.ShapeDtypeStruct((_NC, _HP, _D), jnp.float32),
    mesh=_mesh,
    scratch_types=[
        pltpu.VMEM((5, _CH), jnp.int32),
        pltpu.VMEM((5, _CH), jnp.int32),
        pltpu.VMEM((_CH,), jnp.int32),
        pltpu.VMEM((_CH,), jnp.int32),
        pltpu.VMEM((_CH,), jnp.int32),
        pltpu.VMEM((_CH,), jnp.int32),
        pltpu.VMEM((_CH, _GW), jnp.float32),
        pltpu.VMEM((_CH, _GW), jnp.float32),
        pltpu.VMEM((_CH, _D), jnp.float32),
        pltpu.VMEM((_CH, _D), jnp.float32),
        pltpu.VMEM_SHARED((_HP, _D), jnp.float32),
        pltpu.SemaphoreType.DMA,
        pltpu.SemaphoreType.DMA,
        pltpu.SemaphoreType.DMA,
        pltpu.SemaphoreType.DMA,
        pltpu.SemaphoreType.DMA,
        pltpu.SemaphoreType.DMA,
    ],
)(_main_body)


def _final_body(p_ref, xs_ref, root_ref, dinv_ref, out_ref):
    dinv = dinv_ref[...]
    selfv = jnp.maximum(xs_ref[...] + root_ref[...], 0.0) * (dinv * dinv)
    out_ref[...] = p_ref[...] * dinv + selfv


def _final_call(pfull, xs, root, dinv2):
    return pl.pallas_call(
        _final_body,
        grid=(_N // _R,),
        in_specs=[
            pl.BlockSpec((_R, _D), lambda i: (i, 0)),
            pl.BlockSpec((_R, _D), lambda i: (i, 0)),
            pl.BlockSpec((1, _D), lambda i: (0, 0)),
            pl.BlockSpec((_R, 1), lambda i: (i, 0)),
        ],
        out_specs=pl.BlockSpec((_R, _D), lambda i: (i, 0)),
        out_shape=jax.ShapeDtypeStruct((_N, _D), jnp.float32),
    )(pfull, xs, root, dinv2)


def kernel(x, edge_index, edge_attr, W, b, root_emb, B0, B1, B2):
    row = edge_index[0]
    col = edge_index[1]
    packed = jnp.stack(
        [row, col, edge_attr[:, 0], edge_attr[:, 1], edge_attr[:, 2]], 0)
    packed = packed.reshape(5, _NBK, _CH).transpose(1, 0, 2)

    degp = _deg_call(packed)
    dps = (degp[0, :_N] + degp[1, :_N]).reshape(_N, 1)

    xs, dinv2, C, xgd = _dense_call(
        x, W.T, b.reshape(1, _D), dps, B0, B1, B2)

    parts = _main_call(packed, xgd, C)
    pfull = jnp.concatenate(
        [parts[0, :_HALF], parts[1, :_N - _HALF]], axis=0)

    return _final_call(pfull, xs, root_emb, dinv2)
